# Initial kernel scaffold; baseline (speedup 1.0000x reference)
import functools

import jax
import jax.numpy as jnp
from jax import lax
from jax.experimental import pallas as pl
from jax.experimental.pallas import tpu as pltpu
from jax.experimental.pallas import tpu_sc as plsc

N = 10000
E = 320000
D_IN = 128
HID = 128
H1 = 8
OUT = 64
NEG = 0.2

NP = 10240
SUB = 320
KE = 80
EPT = E // 16
NBLK = EPT // KE

_MESH = dict(core_axis_name="c", subcore_axis_name="s")


def _mm1_body(x_ref, w_ref, o_ref):
    o_ref[0] = jnp.dot(x_ref[...], w_ref[0],
                       preferred_element_type=jnp.float32,
                       precision=lax.Precision.HIGHEST)


def _mm1(x, w9):
    bn = 400
    return pl.pallas_call(
        _mm1_body,
        grid=(9, N // bn),
        in_specs=[
            pl.BlockSpec((bn, D_IN), lambda h, i: (i, 0)),
            pl.BlockSpec((1, D_IN, 128), lambda h, i: (h, 0, 0)),
        ],
        out_specs=pl.BlockSpec((1, bn, 128), lambda h, i: (h, i, 0)),
        out_shape=jax.ShapeDtypeStruct((9, N, 128), jnp.float32),
    )(x, w9)


def _mm2_body(h_ref, w_ref, o_ref):
    acc = jnp.zeros((400, 128), jnp.float32)
    for hh in range(H1):
        a = h_ref[hh]
        a = jnp.where(a > 0, a, jnp.exp(a) - 1.0)
        acc = acc + jnp.dot(a, w_ref[hh],
                            preferred_element_type=jnp.float32,
                            precision=lax.Precision.HIGHEST)
    o_ref[...] = acc


def _mm2(h1, w2e):
    bn = 400
    return pl.pallas_call(
        _mm2_body,
        grid=(N // bn,),
        in_specs=[
            pl.BlockSpec((H1, bn, 128), lambda i: (0, i, 0)),
            pl.BlockSpec((H1, 128, 128), lambda i: (0, 0, 0)),
        ],
        out_specs=pl.BlockSpec((bn, 128), lambda i: (i, 0)),
        out_shape=jax.ShapeDtypeStruct((N, 128), jnp.float32),
    )(h1, w2e)


def _edge_sweep(src_h, dst_h, feat_h, tbl_off, d, el_t, er_t, s_sh, out_sh,
                src_b, dst_b, w_b, rows, sem):
    s_id = lax.axis_index("s")
    tile_base = s_id * EPT
    nv = KE // 16
    ndv = d // 16

    def blk(j, _):
        ebase = tile_base + j * KE
        pltpu.sync_copy(src_h.at[pl.ds(ebase, KE)], src_b)
        pltpu.sync_copy(dst_h.at[pl.ds(ebase, KE)], dst_b)
        for v in range(nv):
            sl = pl.ds(v * 16, 16)
            s16 = src_b[sl]
            d16 = dst_b[sl]
            el16 = plsc.load_gather(el_t, [s16])
            er16 = plsc.load_gather(er_t, [d16])
            e16 = el16 + er16
            w16 = jnp.exp(jnp.maximum(e16, NEG * e16))
            w_b[sl] = w16
            src_b[sl] = s16 + tbl_off
        pltpu.async_copy(feat_h.at[src_b], rows, sem).wait()

        def scale(r, _):
            a = w_b[r]
            for v in range(ndv):
                sl = pl.ds(v * 16, 16)
                rows[r, sl] = rows[r, sl] * a
            return 0

        lax.fori_loop(0, KE, scale, 0)
        pltpu.sync_copy(w_b, s_sh.at[dst_b], add=True)
        pltpu.sync_copy(rows, out_sh.at[dst_b], add=True)
        return 0

    lax.fori_loop(0, NBLK, blk, 0)


def _finalize(out_h, out_off, d, s_sh, out_sh, sch, inv, och):
    s_id = lax.axis_index("s")
    ndv = d // 16
    for half in range(2):
        c0 = s_id + half * 16
        pltpu.sync_copy(s_sh.at[pl.ds(c0 * SUB, SUB)], sch)
        pltpu.sync_copy(out_sh.at[pl.ds(c0 * SUB, SUB)], och)
        for v in range(SUB // 16):
            sl = pl.ds(v * 16, 16)
            inv[sl] = 1.0 / (sch[sl] + 1e-9)

        def scale(r, _):
            a = inv[r]
            for v in range(ndv):
                sl = pl.ds(v * 16, 16)
                och[r, sl] = och[r, sl] * a
            return 0

        lax.fori_loop(0, SUB, scale, 0)
        pltpu.sync_copy(och, out_h.at[pl.ds(out_off + c0 * SUB, SUB)])


def _sc1_body(src_h, dst_h, elr_h, feat_h, z1_h, z2_h, out_h,
              el_t, er_t, src_b, dst_b, w_b, rows, sch, inv, och,
              s_sh, out_sh, sem):
    c = lax.axis_index("c")
    s_id = lax.axis_index("s")
    for hh in range(4):
        h = c * 4 + hh
        pltpu.sync_copy(z1_h, s_sh.at[pl.ds(s_id * (NP // 16), NP // 16)])
        for half in range(2):
            c0 = s_id + half * 16
            pltpu.sync_copy(z2_h, out_sh.at[pl.ds(c0 * SUB, SUB)])
        pltpu.sync_copy(elr_h.at[pl.ds(h * N, N)], el_t)
        pltpu.sync_copy(elr_h.at[pl.ds((h + 8) * N, N)], er_t)
        plsc.subcore_barrier()
        _edge_sweep(src_h, dst_h, feat_h, h * N, 128, el_t, er_t,
                    s_sh, out_sh, src_b, dst_b, w_b, rows, sem)
        plsc.subcore_barrier()
        _finalize(out_h, h * NP, 128, s_sh, out_sh, sch, inv, och)
        plsc.subcore_barrier()


def _sc1(src, dst, elr, feat, z1, z2):
    kern = pl.kernel(
        _sc1_body,
        mesh=plsc.VectorSubcoreMesh(**_MESH),
        out_type=jax.ShapeDtypeStruct((H1 * NP, 128), jnp.float32),
        scratch_types=[
            pltpu.VMEM((N,), jnp.float32),
            pltpu.VMEM((N,), jnp.float32),
            pltpu.VMEM((KE,), jnp.int32),
            pltpu.VMEM((KE,), jnp.int32),
            pltpu.VMEM((KE,), jnp.float32),
            pltpu.VMEM((KE, 128), jnp.float32),
            pltpu.VMEM((SUB,), jnp.float32),
            pltpu.VMEM((SUB,), jnp.float32),
            pltpu.VMEM((SUB, 128), jnp.float32),
            pltpu.VMEM_SHARED((NP,), jnp.float32),
            pltpu.VMEM_SHARED((NP, 128), jnp.float32),
            pltpu.SemaphoreType.DMA,
        ],
    )
    return kern(src, dst, elr, feat, z1, z2)


def _sc2_body(src_h, dst_h, elr2_h, feat2_h, z1_h, z2_h, out_h,
              el_t, er_t, src_b, dst_b, w_b, rows, sch, inv, och,
              s_sh, out_sh, sem):
    c = lax.axis_index("c")
    s_id = lax.axis_index("s")
    pltpu.sync_copy(z1_h, s_sh.at[pl.ds(s_id * (NP // 16), NP // 16)])
    for half in range(2):
        c0 = s_id + half * 16
        pltpu.sync_copy(z2_h, out_sh.at[pl.ds(c0 * SUB, SUB)])
    pltpu.sync_copy(elr2_h.at[pl.ds(0, N)], el_t)
    pltpu.sync_copy(elr2_h.at[pl.ds(N, N)], er_t)
    plsc.subcore_barrier()
    _edge_sweep(src_h, dst_h, feat2_h, c * N, 32, el_t, er_t,
                s_sh, out_sh, src_b, dst_b, w_b, rows, sem)
    plsc.subcore_barrier()
    _finalize(out_h, c * NP, 32, s_sh, out_sh, sch, inv, och)


def _sc2(src, dst, elr2, feat2h, z1, z2):
    kern = pl.kernel(
        _sc2_body,
        mesh=plsc.VectorSubcoreMesh(**_MESH),
        out_type=jax.ShapeDtypeStruct((2 * NP, 32), jnp.float32),
        scratch_types=[
            pltpu.VMEM((N,), jnp.float32),
            pltpu.VMEM((N,), jnp.float32),
            pltpu.VMEM((KE,), jnp.int32),
            pltpu.VMEM((KE,), jnp.int32),
            pltpu.VMEM((KE,), jnp.float32),
            pltpu.VMEM((KE, 32), jnp.float32),
            pltpu.VMEM((SUB,), jnp.float32),
            pltpu.VMEM((SUB,), jnp.float32),
            pltpu.VMEM((SUB, 32), jnp.float32),
            pltpu.VMEM_SHARED((NP,), jnp.float32),
            pltpu.VMEM_SHARED((NP, 32), jnp.float32),
            pltpu.SemaphoreType.DMA,
        ],
    )
    return kern(src, dst, elr2, feat2h, z1, z2)


def kernel(x, edge_index, W1, aL1, aR1, W2, aL2, aR2):
    src = edge_index[0]
    dst = edge_index[1]

    w1r = W1.reshape(D_IN, H1, HID).transpose(1, 0, 2)
    ul1 = jnp.einsum("hdk,hk->dh", w1r, aL1)
    ur1 = jnp.einsum("hdk,hk->dh", w1r, aR1)
    ulur = jnp.concatenate([ul1, ur1, jnp.zeros((D_IN, 112), jnp.float32)], 1)
    w9 = jnp.concatenate([w1r, ulur[None]], 0)

    f1 = _mm1(x, w9)
    feat_flat = f1.reshape(9 * N, 128)
    elr = f1[8, :, :16].T.reshape(-1)

    z1 = jnp.zeros((NP // 16,), jnp.float32)
    z2 = jnp.zeros((SUB, 128), jnp.float32)
    u1 = _sc1(src, dst, elr, feat_flat, z1, z2)

    w2r = W2.reshape(H1, HID, OUT)
    ul2 = (W2 @ aL2[0]).reshape(H1, HID, 1)
    ur2 = (W2 @ aR2[0]).reshape(H1, HID, 1)
    w2e = jnp.concatenate(
        [w2r, ul2, ur2, jnp.zeros((H1, HID, 62), jnp.float32)], 2)

    m2 = _mm2(u1.reshape(H1, NP, 128), w2e)
    feat2h = jnp.concatenate([m2[:, :32], m2[:, 32:64]], 0)
    elr2 = jnp.concatenate([m2[:, 64], m2[:, 65]], 0)

    z2b = jnp.zeros((SUB, 32), jnp.float32)
    u2 = _sc2(src, dst, elr2, feat2h, z1, z2b)
    o = u2.reshape(2, NP, 32)
    return jnp.concatenate([o[0, :N], o[1, :N]], axis=1)


# speedup vs baseline: 13.9159x; 13.9159x over previous
import functools

import jax
import jax.numpy as jnp
from jax import lax
from jax.experimental import pallas as pl
from jax.experimental.pallas import tpu as pltpu
from jax.experimental.pallas import tpu_sc as plsc

N = 10000
E = 320000
D_IN = 128
HID = 128
H1 = 8
OUT = 64
NEG = 0.2

NP = 10240
SUB = 128
NSUB = NP // SUB // 16
KE = 80
EPT = E // 16
NBLK = EPT // KE

_MESH = dict(core_axis_name="c", subcore_axis_name="s")


def _mm1_body(x_ref, w_ref, o_ref):
    o_ref[0] = jnp.dot(x_ref[...], w_ref[0],
                       preferred_element_type=jnp.float32,
                       precision=lax.Precision.HIGHEST)


def _mm1(x, w9):
    bn = 400
    return pl.pallas_call(
        _mm1_body,
        grid=(9, N // bn),
        in_specs=[
            pl.BlockSpec((bn, D_IN), lambda h, i: (i, 0)),
            pl.BlockSpec((1, D_IN, 128), lambda h, i: (h, 0, 0)),
        ],
        out_specs=pl.BlockSpec((1, bn, 128), lambda h, i: (h, i, 0)),
        out_shape=jax.ShapeDtypeStruct((9, N, 128), jnp.float32),
    )(x, w9)


def _mm2_body(h_ref, w_ref, o_ref):
    acc = jnp.zeros((400, 128), jnp.float32)
    for hh in range(H1):
        a = h_ref[hh]
        a = jnp.where(a > 0, a, jnp.exp(a) - 1.0)
        acc = acc + jnp.dot(a, w_ref[hh],
                            preferred_element_type=jnp.float32,
                            precision=lax.Precision.HIGHEST)
    o_ref[...] = acc


def _mm2(h1, w2e):
    bn = 400
    return pl.pallas_call(
        _mm2_body,
        grid=(N // bn,),
        in_specs=[
            pl.BlockSpec((H1, bn, 128), lambda i: (0, i, 0)),
            pl.BlockSpec((H1, 128, 128), lambda i: (0, 0, 0)),
        ],
        out_specs=pl.BlockSpec((bn, 128), lambda i: (i, 0)),
        out_shape=jax.ShapeDtypeStruct((N, 128), jnp.float32),
    )(h1, w2e)


def _edge_sweep(src_h, dst_h, feat_h, tbl_off, tile_base, nblk,
                el_t, er_t, s_sh, out_sh, src_b, dst_b, w_b, rows, sem):
    nv = KE // 16
    ndv = 8

    def blk(j, _):
        ebase = tile_base + j * KE
        pltpu.sync_copy(src_h.at[pl.ds(ebase, KE)], src_b)
        pltpu.sync_copy(dst_h.at[pl.ds(ebase, KE)], dst_b)
        for v in range(nv):
            sl = pl.ds(v * 16, 16)
            s16 = src_b[sl]
            d16 = dst_b[sl]
            el16 = plsc.load_gather(el_t, [s16])
            er16 = plsc.load_gather(er_t, [d16])
            e16 = el16 + er16
            w16 = jnp.exp(jnp.maximum(e16, NEG * e16))
            w_b[sl] = w16
            if tbl_off is not None:
                src_b[sl] = s16 + tbl_off
        pltpu.async_copy(feat_h.at[src_b], rows, sem).wait()

        def scale(g, _):
            base = g * 16
            w16 = w_b[pl.ds(base, 16)]
            for r0 in range(16):
                a = w16[r0]
                for v in range(ndv):
                    sl = pl.ds(v * 16, 16)
                    rows[base + r0, sl] = rows[base + r0, sl] * a
            return 0

        lax.fori_loop(0, KE // 16, scale, 0)
        pltpu.sync_copy(w_b, s_sh.at[dst_b], add=True)
        pltpu.sync_copy(rows, out_sh.at[dst_b], add=True)
        return 0

    lax.fori_loop(0, nblk, blk, 0)


def _finalize(out_h, out_off, d, s_sh, out_sh, sch, inv, och):
    s_id = lax.axis_index("s")
    ndv = d // 16
    for half in range(NSUB):
        c0 = s_id + half * 16
        pltpu.sync_copy(s_sh.at[pl.ds(c0 * SUB, SUB)], sch)
        pltpu.sync_copy(out_sh.at[pl.ds(c0 * SUB, SUB)], och)
        for v in range(SUB // 16):
            sl = pl.ds(v * 16, 16)
            inv[sl] = 1.0 / (sch[sl] + 1e-9)

        def scale(g, _):
            base = g * 16
            i16 = inv[pl.ds(base, 16)]
            for r0 in range(16):
                a = i16[r0]
                for v in range(ndv):
                    sl = pl.ds(v * 16, 16)
                    och[base + r0, sl] = och[base + r0, sl] * a
            return 0

        lax.fori_loop(0, SUB // 16, scale, 0)
        pltpu.sync_copy(och, out_h.at[pl.ds(out_off + c0 * SUB, SUB)])


def _sc1_body(src_h, dst_h, elr_h, feat_h, z1_h, z2_h, out_h,
              el_t, er_t, src_b, dst_b, w_b, rows, sch, inv, och,
              s_sh, out_sh, sem):
    c = lax.axis_index("c")
    s_id = lax.axis_index("s")
    for hh in range(4):
        h = c * 4 + hh
        pltpu.sync_copy(z1_h, s_sh.at[pl.ds(s_id * (NP // 16), NP // 16)])
        for half in range(NSUB):
            c0 = s_id + half * 16
            pltpu.sync_copy(z2_h, out_sh.at[pl.ds(c0 * SUB, SUB)])
        pltpu.sync_copy(elr_h.at[pl.ds(h * N, N)], el_t)
        pltpu.sync_copy(elr_h.at[pl.ds((h + 8) * N, N)], er_t)
        plsc.subcore_barrier()
        _edge_sweep(src_h, dst_h, feat_h, h * N, s_id * EPT, NBLK,
                    el_t, er_t, s_sh, out_sh, src_b, dst_b, w_b, rows, sem)
        plsc.subcore_barrier()
        _finalize(out_h, h * NP, 128, s_sh, out_sh, sch, inv, och)
        plsc.subcore_barrier()


def _sc1(src, dst, elr, feat, z1, z2):
    kern = pl.kernel(
        _sc1_body,
        mesh=plsc.VectorSubcoreMesh(**_MESH),
        compiler_params=pltpu.CompilerParams(needs_layout_passes=False),
        out_type=jax.ShapeDtypeStruct((H1 * NP, 128), jnp.float32),
        scratch_types=[
            pltpu.VMEM((N,), jnp.float32),
            pltpu.VMEM((N,), jnp.float32),
            pltpu.VMEM((KE,), jnp.int32),
            pltpu.VMEM((KE,), jnp.int32),
            pltpu.VMEM((KE,), jnp.float32),
            pltpu.VMEM((KE, 128), jnp.float32),
            pltpu.VMEM((SUB,), jnp.float32),
            pltpu.VMEM((SUB,), jnp.float32),
            pltpu.VMEM((SUB, 128), jnp.float32),
            pltpu.VMEM_SHARED((NP,), jnp.float32),
            pltpu.VMEM_SHARED((NP, 128), jnp.float32),
            pltpu.SemaphoreType.DMA,
        ],
    )
    return kern(src, dst, elr, feat, z1, z2)


def _sc2_body(src_h, dst_h, elr2_h, feat2_h, z1_h, z2_h, u_h, s_out_h,
              el_t, er_t, src_b, dst_b, w_b, rows,
              s_sh, out_sh, sem):
    c = lax.axis_index("c")
    s_id = lax.axis_index("s")
    chunk = NP // 16
    pltpu.sync_copy(z1_h, s_sh.at[pl.ds(s_id * chunk, chunk)])
    for half in range(NSUB):
        c0 = s_id + half * 16
        pltpu.sync_copy(z2_h, out_sh.at[pl.ds(c0 * SUB, SUB)])
    pltpu.sync_copy(elr2_h.at[pl.ds(0, N)], el_t)
    pltpu.sync_copy(elr2_h.at[pl.ds(N, N)], er_t)
    plsc.subcore_barrier()
    wid = c * 16 + s_id
    _edge_sweep(src_h, dst_h, feat2_h, None, wid * (E // 32), E // 32 // KE,
                el_t, er_t, s_sh, out_sh, src_b, dst_b, w_b, rows, sem)
    plsc.subcore_barrier()
    pltpu.sync_copy(out_sh.at[pl.ds(s_id * chunk, chunk)],
                    u_h.at[pl.ds(c * NP + s_id * chunk, chunk)])
    pltpu.sync_copy(s_sh.at[pl.ds(s_id * chunk, chunk)],
                    s_out_h.at[pl.ds(c * NP + s_id * chunk, chunk)])


def _sc2(src, dst, elr2, feat2, z1, z2):
    kern = pl.kernel(
        _sc2_body,
        mesh=plsc.VectorSubcoreMesh(**_MESH),
        compiler_params=pltpu.CompilerParams(needs_layout_passes=False),
        out_type=[
            jax.ShapeDtypeStruct((2 * NP, 128), jnp.float32),
            jax.ShapeDtypeStruct((2 * NP,), jnp.float32),
        ],
        scratch_types=[
            pltpu.VMEM((N,), jnp.float32),
            pltpu.VMEM((N,), jnp.float32),
            pltpu.VMEM((KE,), jnp.int32),
            pltpu.VMEM((KE,), jnp.int32),
            pltpu.VMEM((KE,), jnp.float32),
            pltpu.VMEM((KE, 128), jnp.float32),
            pltpu.VMEM_SHARED((NP,), jnp.float32),
            pltpu.VMEM_SHARED((NP, 128), jnp.float32),
            pltpu.SemaphoreType.DMA,
        ],
    )
    return kern(src, dst, elr2, feat2, z1, z2)


def _comb_body(u_ref, s_ref, o_ref):
    bn = u_ref.shape[1]
    su = s_ref[0].reshape(bn) + s_ref[1].reshape(bn) + 1e-9
    o_ref[...] = (u_ref[0, :, :OUT] + u_ref[1, :, :OUT]) / su[:, None]


def _combine(u, s):
    bn = 1024
    return pl.pallas_call(
        _comb_body,
        grid=(NP // bn,),
        in_specs=[
            pl.BlockSpec((2, bn, 128), lambda i: (0, i, 0)),
            pl.BlockSpec((2, bn // 128, 128), lambda i: (0, i, 0)),
        ],
        out_specs=pl.BlockSpec((bn, OUT), lambda i: (i, 0)),
        out_shape=jax.ShapeDtypeStruct((NP, OUT), jnp.float32),
    )(u, s)


def kernel(x, edge_index, W1, aL1, aR1, W2, aL2, aR2):
    src = edge_index[0]
    dst = edge_index[1]

    w1r = W1.reshape(D_IN, H1, HID).transpose(1, 0, 2)
    ul1 = jnp.einsum("hdk,hk->dh", w1r, aL1)
    ur1 = jnp.einsum("hdk,hk->dh", w1r, aR1)
    ulur = jnp.concatenate([ul1, ur1, jnp.zeros((D_IN, 112), jnp.float32)], 1)
    w9 = jnp.concatenate([w1r, ulur[None]], 0)

    f1 = _mm1(x, w9)
    feat_flat = f1.reshape(9 * N, 128)
    elr = f1[8, :, :16].T.reshape(-1)

    z1 = jnp.zeros((NP // 16,), jnp.float32)
    z2 = jnp.zeros((SUB, 128), jnp.float32)
    u1 = _sc1(src, dst, elr, feat_flat, z1, z2)

    w2r = W2.reshape(H1, HID, OUT)
    ul2 = (W2 @ aL2[0]).reshape(H1, HID, 1)
    ur2 = (W2 @ aR2[0]).reshape(H1, HID, 1)
    w2e = jnp.concatenate(
        [w2r, ul2, ur2, jnp.zeros((H1, HID, 62), jnp.float32)], 2)

    m2 = _mm2(u1.reshape(H1, NP, 128), w2e)
    elr2 = jnp.concatenate([m2[:, 64], m2[:, 65]], 0)

    u2, s2 = _sc2(src, dst, elr2, m2, z1, z2)
    logits = _combine(u2.reshape(2, NP, 128),
                      s2.reshape(2, NP // 128, 128))
    return logits[:N]


# speedup vs baseline: 26.9642x; 1.9377x over previous
import jax
import jax.numpy as jnp
from jax import lax
from jax.experimental import pallas as pl
from jax.experimental.pallas import tpu as pltpu
from jax.experimental.pallas import tpu_sc as plsc

N = 10000
E = 320000
D_IN = 128
HID = 128
H1 = 8
OUT = 64
NEG = 0.2

NP = 10240
KE = 80
EPT = E // 16
CHB = 4000
NCH = EPT // CHB
BPC = CHB // KE
EPT2 = E // 32
NB2 = EPT2 // KE

_MESH = dict(core_axis_name="c", subcore_axis_name="s")


def _mm1_body(x_ref, w_ref, o_ref):
    o_ref[0] = jnp.dot(x_ref[...], w_ref[0],
                       preferred_element_type=jnp.float32,
                       precision=lax.Precision.HIGHEST)


def _mm1(x, w9):
    bn = 400
    return pl.pallas_call(
        _mm1_body,
        grid=(9, N // bn),
        in_specs=[
            pl.BlockSpec((bn, D_IN), lambda h, i: (i, 0)),
            pl.BlockSpec((1, D_IN, 128), lambda h, i: (h, 0, 0)),
        ],
        out_specs=pl.BlockSpec((1, bn, 128), lambda h, i: (h, i, 0)),
        out_shape=jax.ShapeDtypeStruct((9, N, 128), jnp.float32),
    )(x, w9)


def _mm2_body(h_ref, w_ref, o_ref):
    acc = jnp.zeros((400, 128), jnp.float32)
    for hh in range(H1):
        a = h_ref[hh]
        a = jnp.where(a > 0, a, jnp.exp(a) - 1.0)
        acc = acc + jnp.dot(a, w_ref[hh],
                            preferred_element_type=jnp.float32,
                            precision=lax.Precision.HIGHEST)
    o_ref[...] = acc


def _mm2(h1, w2e):
    bn = 400
    return pl.pallas_call(
        _mm2_body,
        grid=(N // bn,),
        in_specs=[
            pl.BlockSpec((H1, bn, 128), lambda i: (0, i, 0)),
            pl.BlockSpec((H1, 128, 128), lambda i: (0, 0, 0)),
        ],
        out_specs=pl.BlockSpec((bn, 128), lambda i: (i, 0)),
        out_shape=jax.ShapeDtypeStruct((N, 128), jnp.float32),
    )(h1, w2e)


def _comb_body(u_ref, s_ref, o_ref):
    bn = u_ref.shape[1]
    su = s_ref[0].reshape(bn) + s_ref[1].reshape(bn) + 1e-9
    o_ref[...] = (u_ref[0, :, :OUT] + u_ref[1, :, :OUT]) / su[:, None]


def _combine(u, s):
    bn = 1024
    return pl.pallas_call(
        _comb_body,
        grid=(NP // bn,),
        in_specs=[
            pl.BlockSpec((2, bn, 128), lambda i: (0, i, 0)),
            pl.BlockSpec((2, bn // 128, 128), lambda i: (0, i, 0)),
        ],
        out_specs=pl.BlockSpec((bn, OUT), lambda i: (i, 0)),
        out_shape=jax.ShapeDtypeStruct((NP, OUT), jnp.float32),
    )(u, s)


def _stage_a(b, el_off, er_off, src_ch, dst_ch, feat_h, elr_sh,
             buf, wait_scatter, s_sh, out_sh):
    src_gb, dst_gb, dst_sb, elv, erv, w_b, rows, sems = buf
    if wait_scatter is True:
        pltpu.make_async_copy(w_b, s_sh.at[dst_sb], sems[3]).wait()
        pltpu.make_async_copy(rows, out_sh.at[dst_sb], sems[4]).wait()
    elif wait_scatter is not None:
        @pl.when(wait_scatter)
        def _():
            pltpu.make_async_copy(w_b, s_sh.at[dst_sb], sems[3]).wait()
            pltpu.make_async_copy(rows, out_sh.at[dst_sb], sems[4]).wait()
    for v in range(KE // 16):
        sl = pl.ds(v * 16, 16)
        s16 = src_ch[pl.ds(b * KE + v * 16, 16)]
        d16 = dst_ch[pl.ds(b * KE + v * 16, 16)]
        src_gb[sl] = s16 + el_off
        dst_gb[sl] = d16 + er_off
        dst_sb[sl] = d16
    pltpu.async_copy(feat_h.at[src_gb], rows, sems[0])
    pltpu.async_copy(elr_sh.at[src_gb], elv, sems[1])
    pltpu.async_copy(elr_sh.at[dst_gb], erv, sems[2])


def _stage_b(feat_h, elr_sh, buf, s_sh, out_sh):
    src_gb, dst_gb, dst_sb, elv, erv, w_b, rows, sems = buf
    pltpu.make_async_copy(elr_sh.at[src_gb], elv, sems[1]).wait()
    pltpu.make_async_copy(elr_sh.at[dst_gb], erv, sems[2]).wait()
    for v in range(KE // 16):
        sl = pl.ds(v * 16, 16)
        e16 = (plsc.bitcast(elv[sl], jnp.float32)
               + plsc.bitcast(erv[sl], jnp.float32))
        w_b[sl] = jnp.exp(jnp.maximum(e16, NEG * e16))
    pltpu.make_async_copy(feat_h.at[src_gb], rows, sems[0]).wait()

    def scale_row(r, _):
        a = plsc.load_gather(w_b, [jnp.full((16,), r, jnp.int32)])
        for v in range(8):
            sl = pl.ds(v * 16, 16)
            rows[r, sl] = rows[r, sl] * a
        return 0

    lax.fori_loop(0, KE, scale_row, 0)
    pltpu.async_copy(w_b, s_sh.at[dst_sb], sems[3], add=True)
    pltpu.async_copy(rows, out_sh.at[dst_sb], sems[4], add=True)


def _drain_scatters(bufs, s_sh, out_sh):
    for buf in bufs:
        _, _, dst_sb, _, _, w_b, rows, sems = buf
        pltpu.make_async_copy(w_b, s_sh.at[dst_sb], sems[3]).wait()
        pltpu.make_async_copy(rows, out_sh.at[dst_sb], sems[4]).wait()


def _zero_rows_buf(rw):
    def st(g, _):
        for v in range(8):
            rw[g, pl.ds(v * 16, 16)] = jnp.zeros((16,), jnp.float32)
        return 0
    lax.fori_loop(0, KE, st, 0)


def _zero_accumulators(s_id, zb, rw, s_sh, out_sh):
    for k in range(8):
        pltpu.sync_copy(zb, s_sh.at[pl.ds(s_id * 640 + k * KE, KE)])
        pltpu.sync_copy(rw, out_sh.at[pl.ds(s_id * 640 + k * KE, KE)])


def _divide_chunks(s_id, out_h, out_base, s_sh, out_sh, sbuf, ibuf, rows):
    def one(half, _):
        c0 = s_id + half * 16
        pltpu.sync_copy(s_sh.at[pl.ds(c0 * 64, 64)], sbuf.at[pl.ds(0, 64)])
        pltpu.sync_copy(out_sh.at[pl.ds(c0 * 64, 64)], rows.at[pl.ds(0, 64)])
        for v in range(4):
            sl = pl.ds(v * 16, 16)
            ibuf[sl] = 1.0 / (sbuf[sl] + 1e-9)

        def scale_row(r, _):
            a = plsc.load_gather(ibuf, [jnp.full((16,), r, jnp.int32)])
            for v in range(8):
                sl = pl.ds(v * 16, 16)
                rows[r, sl] = rows[r, sl] * a
            return 0

        lax.fori_loop(0, 64, scale_row, 0)
        pltpu.sync_copy(rows.at[pl.ds(0, 64)],
                        out_h.at[pl.ds(out_base + c0 * 64, 64)])
        return 0

    lax.fori_loop(0, NP // 64 // 16, one, 0)


def _sc1_body(src_h, dst_h, elr_h, feat_h, out_h,
              src_c0, src_c1, dst_c0, dst_c1,
              sg0, dg0, ds0, el0, er0, wb0, rw0,
              sg1, dg1, ds1, el1, er1, wb1, rw1,
              elr_sh, s_sh, out_sh,
              f0, e0, r0, sw0, sr0, f1, e1, r1, sw1, sr1, cs0, cd0, cs1, cd1):
    c = lax.axis_index("c")
    s_id = lax.axis_index("s")
    for off, ln in ((0, CHB), (CHB, CHB), (2 * CHB, N - 2 * CHB)):
        pltpu.sync_copy(elr_h.at[pl.ds(s_id * N + off, ln)],
                        src_c0.at[pl.ds(0, ln)])
        pltpu.sync_copy(src_c0.at[pl.ds(0, ln)],
                        elr_sh.at[pl.ds(s_id * N + off, ln)])
    buf0 = (sg0, dg0, ds0, el0, er0, wb0, rw0, (f0, e0, r0, sw0, sr0))
    buf1 = (sg1, dg1, ds1, el1, er1, wb1, rw1, (f1, e1, r1, sw1, sr1))
    src_ch = (src_c0, src_c1)
    dst_ch = (dst_c0, dst_c1)
    csems = ((cs0, cd0), (cs1, cd1))
    tile_base = s_id * EPT

    def head(hh, _):
        h = c * 4 + hh
        el_off = h * N
        er_off = (h + 8) * N
        _zero_rows_buf(rw0)
        for v in range(KE // 16):
            wb0[pl.ds(v * 16, 16)] = jnp.zeros((16,), jnp.float32)
        _zero_accumulators(s_id, wb0, rw0, s_sh, out_sh)
        pltpu.async_copy(src_h.at[pl.ds(tile_base, CHB)], src_c0, cs0)
        pltpu.async_copy(dst_h.at[pl.ds(tile_base, CHB)], dst_c0, cd0)
        plsc.subcore_barrier()
        for ci in range(NCH):
            par = ci % 2
            sc, dc = src_ch[par], dst_ch[par]
            pltpu.make_async_copy(src_h.at[pl.ds(tile_base, CHB)], sc,
                                  csems[par][0]).wait()
            pltpu.make_async_copy(dst_h.at[pl.ds(tile_base, CHB)], dc,
                                  csems[par][1]).wait()
            if ci + 1 < NCH:
                nb = (ci + 1) % 2
                off = tile_base + (ci + 1) * CHB
                pltpu.async_copy(src_h.at[pl.ds(off, CHB)],
                                 src_ch[nb], csems[nb][0])
                pltpu.async_copy(dst_h.at[pl.ds(off, CHB)],
                                 dst_ch[nb], csems[nb][1])
            _stage_a(0, el_off, er_off, sc, dc, feat_h, elr_sh, buf0,
                     None if ci == 0 else True, s_sh, out_sh)

            def pair(p, _):
                b0 = 2 * p
                _stage_a(b0 + 1, el_off, er_off, sc, dc, feat_h, elr_sh,
                         buf1, (p > 0) if ci == 0 else True, s_sh, out_sh)
                _stage_b(feat_h, elr_sh, buf0, s_sh, out_sh)

                @pl.when(p < BPC // 2 - 1)
                def _():
                    _stage_a(b0 + 2, el_off, er_off, sc, dc, feat_h, elr_sh,
                             buf0, True, s_sh, out_sh)
                _stage_b(feat_h, elr_sh, buf1, s_sh, out_sh)
                return 0

            lax.fori_loop(0, BPC // 2, pair, 0)
        _drain_scatters((buf0, buf1), s_sh, out_sh)
        plsc.subcore_barrier()
        _divide_chunks(s_id, out_h, h * NP, s_sh, out_sh, wb0, wb1, rw0)
        plsc.subcore_barrier()
        return 0

    lax.fori_loop(0, 4, head, 0)


def _sc1(src, dst, elr_i, feat):
    dma = pltpu.SemaphoreType.DMA
    kern = pl.kernel(
        _sc1_body,
        mesh=plsc.VectorSubcoreMesh(**_MESH),
        compiler_params=pltpu.CompilerParams(needs_layout_passes=False),
        out_type=jax.ShapeDtypeStruct((H1 * NP, 128), jnp.float32),
        scratch_types=[
            pltpu.VMEM((CHB,), jnp.int32),
            pltpu.VMEM((CHB,), jnp.int32),
            pltpu.VMEM((CHB,), jnp.int32),
            pltpu.VMEM((CHB,), jnp.int32),
            pltpu.VMEM((KE,), jnp.int32),
            pltpu.VMEM((KE,), jnp.int32),
            pltpu.VMEM((KE,), jnp.int32),
            pltpu.VMEM((KE,), jnp.int32),
            pltpu.VMEM((KE,), jnp.int32),
            pltpu.VMEM((KE,), jnp.float32),
            pltpu.VMEM((KE, 128), jnp.float32),
            pltpu.VMEM((KE,), jnp.int32),
            pltpu.VMEM((KE,), jnp.int32),
            pltpu.VMEM((KE,), jnp.int32),
            pltpu.VMEM((KE,), jnp.int32),
            pltpu.VMEM((KE,), jnp.int32),
            pltpu.VMEM((KE,), jnp.float32),
            pltpu.VMEM((KE, 128), jnp.float32),
            pltpu.VMEM_SHARED((16 * N,), jnp.int32),
            pltpu.VMEM_SHARED((NP,), jnp.float32),
            pltpu.VMEM_SHARED((NP, 128), jnp.float32),
            dma, dma, dma, dma, dma, dma, dma, dma, dma, dma,
            dma, dma, dma, dma,
        ],
    )
    return kern(src, dst, elr_i, feat)


def _sc2_body(src_h, dst_h, elr2_h, feat2_h, u_h, s_out_h,
              src_c0, dst_c0,
              sg0, dg0, ds0, el0, er0, wb0, rw0,
              sg1, dg1, ds1, el1, er1, wb1, rw1,
              elr_sh, s_sh, out_sh,
              f0, e0, r0, sw0, sr0, f1, e1, r1, sw1, sr1):
    c = lax.axis_index("c")
    s_id = lax.axis_index("s")
    chunk = NP // 16
    pltpu.sync_copy(elr2_h.at[pl.ds(s_id * 1280, 1280)],
                    src_c0.at[pl.ds(0, 1280)])
    pltpu.sync_copy(src_c0.at[pl.ds(0, 1280)],
                    elr_sh.at[pl.ds(s_id * 1280, 1280)])
    _zero_rows_buf(rw0)
    for v in range(KE // 16):
        wb0[pl.ds(v * 16, 16)] = jnp.zeros((16,), jnp.float32)
    _zero_accumulators(s_id, wb0, rw0, s_sh, out_sh)
    wid = c * 16 + s_id
    tile_base = wid * EPT2
    pltpu.sync_copy(src_h.at[pl.ds(tile_base, EPT2)], src_c0)
    pltpu.sync_copy(dst_h.at[pl.ds(tile_base, EPT2)], dst_c0)
    plsc.subcore_barrier()
    buf0 = (sg0, dg0, ds0, el0, er0, wb0, rw0, (f0, e0, r0, sw0, sr0))
    buf1 = (sg1, dg1, ds1, el1, er1, wb1, rw1, (f1, e1, r1, sw1, sr1))

    _stage_a(0, 0, N, src_c0, dst_c0, feat2_h, elr_sh, buf0, None,
             s_sh, out_sh)

    def pair(p, _):
        b0 = 2 * p
        _stage_a(b0 + 1, 0, N, src_c0, dst_c0, feat2_h, elr_sh, buf1,
                 p > 0, s_sh, out_sh)
        _stage_b(feat2_h, elr_sh, buf0, s_sh, out_sh)
        _stage_a(b0 + 2, 0, N, src_c0, dst_c0, feat2_h, elr_sh, buf0,
                 True, s_sh, out_sh)
        _stage_b(feat2_h, elr_sh, buf1, s_sh, out_sh)
        return 0

    lax.fori_loop(0, NB2 // 2, pair, 0)
    _stage_b(feat2_h, elr_sh, buf0, s_sh, out_sh)
    _drain_scatters((buf0, buf1), s_sh, out_sh)
    plsc.subcore_barrier()
    pltpu.sync_copy(out_sh.at[pl.ds(s_id * chunk, chunk)],
                    u_h.at[pl.ds(c * NP + s_id * chunk, chunk)])
    pltpu.sync_copy(s_sh.at[pl.ds(s_id * chunk, chunk)],
                    s_out_h.at[pl.ds(c * NP + s_id * chunk, chunk)])


def _sc2(src, dst, elr2_i, feat2):
    dma = pltpu.SemaphoreType.DMA
    kern = pl.kernel(
        _sc2_body,
        mesh=plsc.VectorSubcoreMesh(**_MESH),
        compiler_params=pltpu.CompilerParams(needs_layout_passes=False),
        out_type=[
            jax.ShapeDtypeStruct((2 * NP, 128), jnp.float32),
            jax.ShapeDtypeStruct((2 * NP,), jnp.float32),
        ],
        scratch_types=[
            pltpu.VMEM((EPT2,), jnp.int32),
            pltpu.VMEM((EPT2,), jnp.int32),
            pltpu.VMEM((KE,), jnp.int32),
            pltpu.VMEM((KE,), jnp.int32),
            pltpu.VMEM((KE,), jnp.int32),
            pltpu.VMEM((KE,), jnp.int32),
            pltpu.VMEM((KE,), jnp.int32),
            pltpu.VMEM((KE,), jnp.float32),
            pltpu.VMEM((KE, 128), jnp.float32),
            pltpu.VMEM((KE,), jnp.int32),
            pltpu.VMEM((KE,), jnp.int32),
            pltpu.VMEM((KE,), jnp.int32),
            pltpu.VMEM((KE,), jnp.int32),
            pltpu.VMEM((KE,), jnp.int32),
            pltpu.VMEM((KE,), jnp.float32),
            pltpu.VMEM((KE, 128), jnp.float32),
            pltpu.VMEM_SHARED((20480,), jnp.int32),
            pltpu.VMEM_SHARED((NP,), jnp.float32),
            pltpu.VMEM_SHARED((NP, 128), jnp.float32),
            dma, dma, dma, dma, dma, dma, dma, dma, dma, dma,
        ],
    )
    return kern(src, dst, elr2_i, feat2)


def kernel(x, edge_index, W1, aL1, aR1, W2, aL2, aR2):
    src = edge_index[0]
    dst = edge_index[1]

    w1r = W1.reshape(D_IN, H1, HID).transpose(1, 0, 2)
    ul1 = jnp.einsum("hdk,hk->dh", w1r, aL1)
    ur1 = jnp.einsum("hdk,hk->dh", w1r, aR1)
    ulur = jnp.concatenate([ul1, ur1, jnp.zeros((D_IN, 112), jnp.float32)], 1)
    w9 = jnp.concatenate([w1r, ulur[None]], 0)

    f1 = _mm1(x, w9)
    feat_flat = f1.reshape(9 * N, 128)
    elr = f1[8, :, :16].T.reshape(-1)
    elr_i = lax.bitcast_convert_type(elr, jnp.int32)

    u1 = _sc1(src, dst, elr_i, feat_flat)

    w2r = W2.reshape(H1, HID, OUT)
    ul2 = (W2 @ aL2[0]).reshape(H1, HID, 1)
    ur2 = (W2 @ aR2[0]).reshape(H1, HID, 1)
    w2e = jnp.concatenate(
        [w2r, ul2, ur2, jnp.zeros((H1, HID, 62), jnp.float32)], 2)

    m2 = _mm2(u1.reshape(H1, NP, 128), w2e)
    elr2 = jnp.concatenate(
        [m2[:, 64], m2[:, 65], jnp.zeros((480,), jnp.float32)])
    elr2_i = lax.bitcast_convert_type(elr2, jnp.int32)

    u2, s2 = _sc2(src, dst, elr2_i, m2)
    logits = _combine(u2.reshape(2, NP, 128),
                      s2.reshape(2, NP // 128, 128))
    return logits[:N]
